# baseline (speedup 1.0000x reference)
import functools

import jax
import jax.numpy as jnp
from jax import lax
from jax.experimental import pallas as pl
from jax.experimental.pallas import tpu as pltpu
from jax.experimental.pallas import tpu_sc as plsc

N = 10000
E = 320000
NC = 2
NS = 16
NW = NC * NS
EPW = E // NW
K = 80
NCHUNK = EPW // K
NBUF = 5
NGROUP = NCHUNK // NBUF
NPAD = 10240
RPT = NPAD // NS

_MESH = plsc.VectorSubcoreMesh(core_axis_name="c", subcore_axis_name="s",
                               num_cores=NC, num_subcores=NS)


def _make_edge_agg(D, with_gather):
    def body(g_hbm, srcr_hbm, dstr_hbm, zeros_hbm, out_hbm,
             src_slab, dst_slab, rows, acc, g_sp, gsems, ssems):
        cid = lax.axis_index("c")
        sid = lax.axis_index("s")
        w = cid * NS + sid
        if with_gather:
            pltpu.sync_copy(srcr_hbm.at[w], src_slab)
            @pl.when(sid < NS - 1)
            def _():
                pltpu.sync_copy(g_hbm.at[pl.ds(sid * 624, 624)],
                                g_sp.at[pl.ds(sid * 624, 624)])

            @pl.when(sid == NS - 1)
            def _():
                pltpu.sync_copy(g_hbm.at[pl.ds(9360, 640)],
                                g_sp.at[pl.ds(9360, 640)])
        else:
            pltpu.sync_copy(g_hbm, rows.at[0])
        pltpu.sync_copy(dstr_hbm.at[w], dst_slab)
        pltpu.sync_copy(zeros_hbm.at[pl.ds(sid * RPT, RPT)],
                        acc.at[pl.ds(sid * RPT, RPT)])
        plsc.subcore_barrier()

        if with_gather:
            for b in range(NBUF):
                pltpu.async_copy(g_sp.at[src_slab.at[b]], rows.at[b],
                                 gsems[b])

            @pl.loop(0, NGROUP)
            def _(g):
                j0 = g * NBUF
                scats = []
                for b in range(NBUF):
                    pltpu.make_async_copy(g_sp.at[src_slab.at[j0 + b]],
                                          rows.at[b], gsems[b]).wait()
                    scats.append(pltpu.async_copy(
                        rows.at[b], acc.at[dst_slab.at[j0 + b]],
                        ssems[b], add=True))
                for b in range(NBUF):
                    scats[b].wait()

                    @pl.when(j0 + NBUF + b < NCHUNK)
                    def _():
                        pltpu.async_copy(
                            g_sp.at[src_slab.at[j0 + NBUF + b]],
                            rows.at[b], gsems[b])
        else:
            @pl.loop(0, NGROUP)
            def _(g):
                j0 = g * NBUF
                scats = [pltpu.async_copy(rows.at[0],
                                          acc.at[dst_slab.at[j0 + b]],
                                          ssems[b], add=True)
                         for b in range(NBUF)]
                for s in scats:
                    s.wait()

        plsc.subcore_barrier()
        pltpu.sync_copy(acc.at[pl.ds(sid * RPT, RPT)],
                        out_hbm.at[cid, pl.ds(sid * RPT, RPT)])

    return pl.kernel(
        body,
        out_type=jax.ShapeDtypeStruct((NC, NPAD, D), jnp.float32),
        mesh=_MESH,
        scratch_types=[
            pltpu.VMEM((NCHUNK, K), jnp.int32),
            pltpu.VMEM((NCHUNK, K), jnp.int32),
            pltpu.VMEM((NBUF, K, D), jnp.float32),
            pltpu.VMEM_SHARED((NPAD, D), jnp.float32),
            pltpu.VMEM_SHARED((N, D), jnp.float32),
            [pltpu.SemaphoreType.DMA] * NBUF,
            [pltpu.SemaphoreType.DMA] * NBUF,
        ],
        compiler_params=pltpu.CompilerParams(use_tc_tiling_on_sc=False),
    )


_agg64 = _make_edge_agg(64, True)
_agg8 = _make_edge_agg(8, True)
_deg8 = _make_edge_agg(8, False)

_B = 1000


def _tc_a_body(pdeg_ref, x_ref, w1_ref, g1_ref, dinv_ref):
    deg = pdeg_ref[0, :, 0:1] + pdeg_ref[1, :, 0:1] + 1.0
    dinv = lax.rsqrt(deg)
    dinv_ref[...] = dinv
    g1_ref[...] = lax.dot(x_ref[...] * dinv, w1_ref[...],
                          preferred_element_type=jnp.float32)


def _tc_mid_body(p_ref, g_ref, dinv_ref, b_ref, w_ref, out_ref):
    dinv = dinv_ref[...]
    agg = p_ref[0] + p_ref[1] + g_ref[...]
    h = jnp.maximum(agg * dinv + b_ref[...], 0.0)
    out_ref[...] = lax.dot(h * dinv, w_ref[...],
                           preferred_element_type=jnp.float32)


def _tc_d_body(p_ref, g_ref, dinv_ref, b_ref, out_ref):
    agg = p_ref[0, :, 0:1] + p_ref[1, :, 0:1] + g_ref[:, 0:1]
    out_ref[...] = agg * dinv_ref[...] + b_ref[...]


def _row_block(d):
    return pl.BlockSpec((_B, d), lambda i: (i, 0))


def _part_block(d):
    return pl.BlockSpec((NC, _B, d), lambda i: (0, i, 0))


def _const_block(r, c):
    return pl.BlockSpec((r, c), lambda i: (0, 0))


_GRID = (N // _B,)

_tc_a = pl.pallas_call(
    _tc_a_body,
    grid=_GRID,
    in_specs=[_part_block(8), _row_block(128), _const_block(128, 64)],
    out_specs=[_row_block(64), _row_block(1)],
    out_shape=[jax.ShapeDtypeStruct((N, 64), jnp.float32),
               jax.ShapeDtypeStruct((N, 1), jnp.float32)],
)


def _tc_mid(dout):
    return pl.pallas_call(
        _tc_mid_body,
        grid=_GRID,
        in_specs=[_part_block(64), _row_block(64), _row_block(1),
                  _const_block(1, 64), _const_block(64, dout)],
        out_specs=_row_block(dout),
        out_shape=jax.ShapeDtypeStruct((N, dout), jnp.float32),
    )


_tc_b = _tc_mid(64)
_tc_c = _tc_mid(8)

_tc_d = pl.pallas_call(
    _tc_d_body,
    grid=_GRID,
    in_specs=[_part_block(8), _row_block(8), _row_block(1),
              _const_block(1, 1)],
    out_specs=_row_block(1),
    out_shape=jax.ShapeDtypeStruct((N, 1), jnp.float32),
)


def kernel(x, edge_index, W1, b1, W2, b2, W3, b3):
    src = edge_index[0].astype(jnp.int32)
    dst = edge_index[1].astype(jnp.int32)
    srcr = src.reshape(NW, NCHUNK, K)
    dstr = dst.reshape(NW, NCHUNK, K)
    zeros64 = jnp.zeros((NPAD, 64), jnp.float32)
    zeros8 = jnp.zeros((NPAD, 8), jnp.float32)
    ones8 = jnp.ones((K, 8), jnp.float32)

    pdeg = _deg8(ones8, srcr, dstr, zeros8)
    g1, dinv = _tc_a(pdeg, x, W1)
    p1 = _agg64(g1, srcr, dstr, zeros64)
    g2 = _tc_b(p1, g1, dinv, b1.reshape(1, 64), W2)
    p2 = _agg64(g2, srcr, dstr, zeros64)
    W3_8 = jnp.tile(W3, (1, 8))
    g3 = _tc_c(p2, g2, dinv, b2.reshape(1, 64), W3_8)
    p3 = _agg8(g3, srcr, dstr, zeros8)
    return _tc_d(p3, g3, dinv, b3.reshape(1, 1))


# speedup vs baseline: 1.1797x; 1.1797x over previous
import functools

import jax
import jax.numpy as jnp
from jax import lax
from jax.experimental import pallas as pl
from jax.experimental.pallas import tpu as pltpu
from jax.experimental.pallas import tpu_sc as plsc

N = 10000
E = 320000
NC = 2
NS = 16
NW = NC * NS
EPW = E // NW
K = 200
NCHUNK = EPW // K
NBUF = 5
NGROUP = NCHUNK // NBUF
NPAD = 10240
RPT = NPAD // NS

_MESH = plsc.VectorSubcoreMesh(core_axis_name="c", subcore_axis_name="s",
                               num_cores=NC, num_subcores=NS)


def _make_edge_agg(D, with_gather):
    def body(g_hbm, srcr_hbm, dstr_hbm, zeros_hbm, out_hbm,
             src_slab, dst_slab, rows, acc, gsems, ssems):
        cid = lax.axis_index("c")
        sid = lax.axis_index("s")
        w = cid * NS + sid
        if with_gather:
            pltpu.sync_copy(srcr_hbm.at[w], src_slab)
        else:
            pltpu.sync_copy(g_hbm, rows.at[0])
        pltpu.sync_copy(dstr_hbm.at[w], dst_slab)
        pltpu.sync_copy(zeros_hbm.at[pl.ds(sid * RPT, RPT)],
                        acc.at[pl.ds(sid * RPT, RPT)])
        plsc.subcore_barrier()

        if with_gather:
            for b in range(NBUF):
                pltpu.async_copy(g_hbm.at[src_slab.at[b]], rows.at[b],
                                 gsems[b])

            @pl.loop(0, NGROUP)
            def _(g):
                j0 = g * NBUF
                scats = []
                for b in range(NBUF):
                    pltpu.make_async_copy(g_hbm.at[src_slab.at[j0 + b]],
                                          rows.at[b], gsems[b]).wait()
                    scats.append(pltpu.async_copy(
                        rows.at[b], acc.at[dst_slab.at[j0 + b]],
                        ssems[b], add=True))
                for b in range(NBUF):
                    scats[b].wait()

                    @pl.when(j0 + NBUF + b < NCHUNK)
                    def _():
                        pltpu.async_copy(
                            g_hbm.at[src_slab.at[j0 + NBUF + b]],
                            rows.at[b], gsems[b])
        else:
            @pl.loop(0, NGROUP)
            def _(g):
                j0 = g * NBUF
                scats = [pltpu.async_copy(rows.at[0],
                                          acc.at[dst_slab.at[j0 + b]],
                                          ssems[b], add=True)
                         for b in range(NBUF)]
                for s in scats:
                    s.wait()

        plsc.subcore_barrier()
        pltpu.sync_copy(acc.at[pl.ds(sid * RPT, RPT)],
                        out_hbm.at[cid, pl.ds(sid * RPT, RPT)])

    return pl.kernel(
        body,
        out_type=jax.ShapeDtypeStruct((NC, NPAD, D), jnp.float32),
        mesh=_MESH,
        scratch_types=[
            pltpu.VMEM((NCHUNK, K), jnp.int32),
            pltpu.VMEM((NCHUNK, K), jnp.int32),
            pltpu.VMEM((NBUF, K, D), jnp.float32),
            pltpu.VMEM_SHARED((NPAD, D), jnp.float32),
            [pltpu.SemaphoreType.DMA] * NBUF,
            [pltpu.SemaphoreType.DMA] * NBUF,
        ],
        compiler_params=pltpu.CompilerParams(use_tc_tiling_on_sc=False),
    )


_agg64 = _make_edge_agg(64, True)
_agg8 = _make_edge_agg(8, True)
_deg8 = _make_edge_agg(8, False)

_B = 1000


def _tc_a_body(pdeg_ref, x_ref, w1_ref, g1_ref, dinv_ref):
    deg = pdeg_ref[0, :, 0:1] + pdeg_ref[1, :, 0:1] + 1.0
    dinv = lax.rsqrt(deg)
    dinv_ref[...] = dinv
    g1_ref[...] = lax.dot(x_ref[...] * dinv, w1_ref[...],
                          preferred_element_type=jnp.float32)


def _tc_mid_body(p_ref, g_ref, dinv_ref, b_ref, w_ref, out_ref):
    dinv = dinv_ref[...]
    agg = p_ref[0] + p_ref[1] + g_ref[...]
    h = jnp.maximum(agg * dinv + b_ref[...], 0.0)
    out_ref[...] = lax.dot(h * dinv, w_ref[...],
                           preferred_element_type=jnp.float32)


def _tc_d_body(p_ref, g_ref, dinv_ref, b_ref, out_ref):
    agg = p_ref[0, :, 0:1] + p_ref[1, :, 0:1] + g_ref[:, 0:1]
    out_ref[...] = agg * dinv_ref[...] + b_ref[...]


def _row_block(d):
    return pl.BlockSpec((_B, d), lambda i: (i, 0))


def _part_block(d):
    return pl.BlockSpec((NC, _B, d), lambda i: (0, i, 0))


def _const_block(r, c):
    return pl.BlockSpec((r, c), lambda i: (0, 0))


_GRID = (N // _B,)

_tc_a = pl.pallas_call(
    _tc_a_body,
    grid=_GRID,
    in_specs=[_part_block(8), _row_block(128), _const_block(128, 64)],
    out_specs=[_row_block(64), _row_block(1)],
    out_shape=[jax.ShapeDtypeStruct((N, 64), jnp.float32),
               jax.ShapeDtypeStruct((N, 1), jnp.float32)],
)


def _tc_mid(dout):
    return pl.pallas_call(
        _tc_mid_body,
        grid=_GRID,
        in_specs=[_part_block(64), _row_block(64), _row_block(1),
                  _const_block(1, 64), _const_block(64, dout)],
        out_specs=_row_block(dout),
        out_shape=jax.ShapeDtypeStruct((N, dout), jnp.float32),
    )


_tc_b = _tc_mid(64)
_tc_c = _tc_mid(8)

_tc_d = pl.pallas_call(
    _tc_d_body,
    grid=_GRID,
    in_specs=[_part_block(8), _row_block(8), _row_block(1),
              _const_block(1, 1)],
    out_specs=_row_block(1),
    out_shape=jax.ShapeDtypeStruct((N, 1), jnp.float32),
)


def kernel(x, edge_index, W1, b1, W2, b2, W3, b3):
    src = edge_index[0].astype(jnp.int32)
    dst = edge_index[1].astype(jnp.int32)
    srcr = src.reshape(NW, NCHUNK, K)
    dstr = dst.reshape(NW, NCHUNK, K)
    zeros64 = jnp.zeros((NPAD, 64), jnp.float32)
    zeros8 = jnp.zeros((NPAD, 8), jnp.float32)
    ones8 = jnp.ones((K, 8), jnp.float32)

    pdeg = _deg8(ones8, srcr, dstr, zeros8)
    g1, dinv = _tc_a(pdeg, x, W1)
    p1 = _agg64(g1, srcr, dstr, zeros64)
    g2 = _tc_b(p1, g1, dinv, b1.reshape(1, 64), W2)
    p2 = _agg64(g2, srcr, dstr, zeros64)
    W3_8 = jnp.tile(W3, (1, 8))
    g3 = _tc_c(p2, g2, dinv, b2.reshape(1, 64), W3_8)
    p3 = _agg8(g3, srcr, dstr, zeros8)
    return _tc_d(p3, g3, dinv, b3.reshape(1, 1))


# speedup vs baseline: 1.2627x; 1.0703x over previous
import functools

import jax
import jax.numpy as jnp
from jax import lax
from jax.experimental import pallas as pl
from jax.experimental.pallas import tpu as pltpu
from jax.experimental.pallas import tpu_sc as plsc

N = 10000
E = 320000
NC = 2
NS = 16
NW = NC * NS
EPW = E // NW
K = 200
NCHUNK = EPW // K
NBUF = 5
NGROUP = NCHUNK // NBUF
NPAD = 10240
RPT = NPAD // NS

_MESH = plsc.VectorSubcoreMesh(core_axis_name="c", subcore_axis_name="s",
                               num_cores=NC, num_subcores=NS)


def _make_edge_agg(D, with_gather):
    def body(g_hbm, ei_hbm, zeros_hbm, out_hbm,
             src_slab, dst_slab, rows, acc, gsems, ssems):
        cid = lax.axis_index("c")
        sid = lax.axis_index("s")
        w = cid * NS + sid
        if with_gather:
            pltpu.sync_copy(ei_hbm.at[0, pl.ds(w * EPW, EPW)], src_slab)
        else:
            pltpu.sync_copy(g_hbm, rows.at[0])
        pltpu.sync_copy(ei_hbm.at[1, pl.ds(w * EPW, EPW)], dst_slab)
        pltpu.sync_copy(zeros_hbm.at[pl.ds(sid * RPT, RPT)],
                        acc.at[pl.ds(sid * RPT, RPT)])
        plsc.subcore_barrier()

        if with_gather:
            for b in range(NBUF):
                pltpu.async_copy(g_hbm.at[src_slab.at[pl.ds(b * K, K)]],
                                 rows.at[b], gsems[b])

            @pl.loop(0, NGROUP)
            def _(g):
                j0 = g * NBUF
                scats = []
                for b in range(NBUF):
                    pltpu.make_async_copy(
                        g_hbm.at[src_slab.at[pl.ds((j0 + b) * K, K)]],
                        rows.at[b], gsems[b]).wait()
                    scats.append(pltpu.async_copy(
                        rows.at[b], acc.at[dst_slab.at[pl.ds((j0 + b) * K, K)]],
                        ssems[b], add=True))
                for b in range(NBUF):
                    scats[b].wait()

                    @pl.when(j0 + NBUF + b < NCHUNK)
                    def _():
                        pltpu.async_copy(
                            g_hbm.at[src_slab.at[pl.ds((j0 + NBUF + b) * K, K)]],
                            rows.at[b], gsems[b])
        else:
            @pl.loop(0, NGROUP)
            def _(g):
                j0 = g * NBUF
                scats = [pltpu.async_copy(
                             rows.at[0],
                             acc.at[dst_slab.at[pl.ds((j0 + b) * K, K)]],
                             ssems[b], add=True)
                         for b in range(NBUF)]
                for s in scats:
                    s.wait()

        plsc.subcore_barrier()
        pltpu.sync_copy(acc.at[pl.ds(sid * RPT, RPT)],
                        out_hbm.at[cid, pl.ds(sid * RPT, RPT)])

    return pl.kernel(
        body,
        out_type=jax.ShapeDtypeStruct((NC, NPAD, D), jnp.float32),
        mesh=_MESH,
        scratch_types=[
            pltpu.VMEM((EPW,), jnp.int32),
            pltpu.VMEM((EPW,), jnp.int32),
            pltpu.VMEM((NBUF, K, D), jnp.float32),
            pltpu.VMEM_SHARED((NPAD, D), jnp.float32),
            [pltpu.SemaphoreType.DMA] * NBUF,
            [pltpu.SemaphoreType.DMA] * NBUF,
        ],
        compiler_params=pltpu.CompilerParams(use_tc_tiling_on_sc=False),
    )


_agg64 = _make_edge_agg(64, True)
_agg8 = _make_edge_agg(8, True)
_deg8 = _make_edge_agg(8, False)

_B = 2000


def _tc_a_body(pdeg_ref, x_ref, w1_ref, g1_ref, dinv_ref):
    deg = pdeg_ref[0, :, 0:1] + pdeg_ref[1, :, 0:1] + 1.0
    dinv = lax.rsqrt(deg)
    dinv_ref[...] = dinv
    g1_ref[...] = lax.dot(x_ref[...] * dinv, w1_ref[...],
                          preferred_element_type=jnp.float32)


def _tc_mid_body(p_ref, g_ref, dinv_ref, b_ref, w_ref, out_ref):
    dinv = dinv_ref[...]
    agg = p_ref[0] + p_ref[1] + g_ref[...]
    h = jnp.maximum(agg * dinv + b_ref[...], 0.0)
    out_ref[...] = lax.dot(h * dinv, w_ref[...],
                           preferred_element_type=jnp.float32)


def _tc_d_body(p_ref, g_ref, dinv_ref, b_ref, out_ref):
    agg = p_ref[0, :, 0:1] + p_ref[1, :, 0:1] + g_ref[:, 0:1]
    out_ref[...] = agg * dinv_ref[...] + b_ref[...]


def _row_block(d):
    return pl.BlockSpec((_B, d), lambda i: (i, 0))


def _part_block(d):
    return pl.BlockSpec((NC, _B, d), lambda i: (0, i, 0))


def _const_block(r, c):
    return pl.BlockSpec((r, c), lambda i: (0, 0))


_GRID = (N // _B,)

_tc_a = pl.pallas_call(
    _tc_a_body,
    grid=_GRID,
    in_specs=[_part_block(8), _row_block(128), _const_block(128, 64)],
    out_specs=[_row_block(64), _row_block(1)],
    out_shape=[jax.ShapeDtypeStruct((N, 64), jnp.float32),
               jax.ShapeDtypeStruct((N, 1), jnp.float32)],
)


def _tc_mid(dout):
    return pl.pallas_call(
        _tc_mid_body,
        grid=_GRID,
        in_specs=[_part_block(64), _row_block(64), _row_block(1),
                  _const_block(1, 64), _const_block(64, dout)],
        out_specs=_row_block(dout),
        out_shape=jax.ShapeDtypeStruct((N, dout), jnp.float32),
    )


_tc_b = _tc_mid(64)
_tc_c = _tc_mid(8)

_tc_d = pl.pallas_call(
    _tc_d_body,
    grid=_GRID,
    in_specs=[_part_block(8), _row_block(8), _row_block(1),
              _const_block(1, 1)],
    out_specs=_row_block(1),
    out_shape=jax.ShapeDtypeStruct((N, 1), jnp.float32),
)


def kernel(x, edge_index, W1, b1, W2, b2, W3, b3):
    ei = edge_index.astype(jnp.int32)
    zeros64 = jnp.zeros((NPAD, 64), jnp.float32)
    zeros8 = jnp.zeros((NPAD, 8), jnp.float32)
    ones8 = jnp.ones((K, 8), jnp.float32)

    pdeg = _deg8(ones8, ei, zeros8)
    g1, dinv = _tc_a(pdeg, x, W1)
    p1 = _agg64(g1, ei, zeros64)
    g2 = _tc_b(p1, g1, dinv, b1.reshape(1, 64), W2)
    p2 = _agg64(g2, ei, zeros64)
    W3_8 = jnp.tile(W3, (1, 8))
    g3 = _tc_c(p2, g2, dinv, b2.reshape(1, 64), W3_8)
    p3 = _agg8(g3, ei, zeros8)
    return _tc_d(p3, g3, dinv, b3.reshape(1, 1))


# speedup vs baseline: 1.2742x; 1.0091x over previous
import functools

import jax
import jax.numpy as jnp
from jax import lax
from jax.experimental import pallas as pl
from jax.experimental.pallas import tpu as pltpu
from jax.experimental.pallas import tpu_sc as plsc

N = 10000
E = 320000
NC = 2
NS = 16
NW = NC * NS
EPW = E // NW
K = 200
NCHUNK = EPW // K
NBUF = 5
NGROUP = NCHUNK // NBUF
NPAD = 10240
RPT = NPAD // NS

_MESH = plsc.VectorSubcoreMesh(core_axis_name="c", subcore_axis_name="s",
                               num_cores=NC, num_subcores=NS)


def _make_edge_agg(D, with_gather):
    def body(g_hbm, ei_hbm, zeros_hbm, out_hbm,
             src_slab, dst_slab, rows, acc, gsems, ssems):
        cid = lax.axis_index("c")
        sid = lax.axis_index("s")
        w = cid * NS + sid
        if with_gather:
            pltpu.sync_copy(ei_hbm.at[0, pl.ds(w * EPW, EPW)], src_slab)
        else:
            pltpu.sync_copy(g_hbm, rows.at[0])
        pltpu.sync_copy(ei_hbm.at[1, pl.ds(w * EPW, EPW)], dst_slab)
        pltpu.sync_copy(zeros_hbm.at[pl.ds(sid * RPT, RPT)],
                        acc.at[pl.ds(sid * RPT, RPT)])
        plsc.subcore_barrier()

        if with_gather:
            for b in range(NBUF):
                pltpu.async_copy(g_hbm.at[src_slab.at[pl.ds(b * K, K)]],
                                 rows.at[b], gsems[b])

            @pl.loop(0, NGROUP)
            def _(g):
                j0 = g * NBUF
                scats = []
                for b in range(NBUF):
                    pltpu.make_async_copy(
                        g_hbm.at[src_slab.at[pl.ds((j0 + b) * K, K)]],
                        rows.at[b], gsems[b]).wait()
                    scats.append(pltpu.async_copy(
                        rows.at[b], acc.at[dst_slab.at[pl.ds((j0 + b) * K, K)]],
                        ssems[b], add=True))
                for b in range(NBUF):
                    scats[b].wait()

                    @pl.when(j0 + NBUF + b < NCHUNK)
                    def _():
                        pltpu.async_copy(
                            g_hbm.at[src_slab.at[pl.ds((j0 + NBUF + b) * K, K)]],
                            rows.at[b], gsems[b])
        else:
            @pl.loop(0, NGROUP)
            def _(g):
                j0 = g * NBUF
                scats = [pltpu.async_copy(
                             rows.at[0],
                             acc.at[dst_slab.at[pl.ds((j0 + b) * K, K)]],
                             ssems[b], add=True)
                         for b in range(NBUF)]
                for s in scats:
                    s.wait()

        plsc.subcore_barrier()
        pltpu.sync_copy(acc.at[pl.ds(sid * RPT, RPT)],
                        out_hbm.at[cid, pl.ds(sid * RPT, RPT)])

    return pl.kernel(
        body,
        out_type=jax.ShapeDtypeStruct((NC, NPAD, D), jnp.float32),
        mesh=_MESH,
        scratch_types=[
            pltpu.VMEM((EPW,), jnp.int32),
            pltpu.VMEM((EPW,), jnp.int32),
            pltpu.VMEM((NBUF, K, D), jnp.float32),
            pltpu.VMEM_SHARED((NPAD, D), jnp.float32),
            [pltpu.SemaphoreType.DMA] * NBUF,
            [pltpu.SemaphoreType.DMA] * NBUF,
        ],
        compiler_params=pltpu.CompilerParams(use_tc_tiling_on_sc=False),
    )


_agg64 = _make_edge_agg(64, True)
_agg8 = _make_edge_agg(8, True)
_deg8 = _make_edge_agg(8, False)

_B = 5000


def _tc_a_body(pdeg_ref, x_ref, w1_ref, g1_ref, dinv_ref):
    deg = pdeg_ref[0, :, 0:1] + pdeg_ref[1, :, 0:1] + 1.0
    dinv = lax.rsqrt(deg)
    dinv_ref[...] = dinv
    g1_ref[...] = lax.dot(x_ref[...] * dinv, w1_ref[...],
                          preferred_element_type=jnp.float32)


def _tc_mid_body(p_ref, g_ref, dinv_ref, b_ref, w_ref, out_ref):
    dinv = dinv_ref[...]
    agg = p_ref[0] + p_ref[1] + g_ref[...]
    h = jnp.maximum(agg * dinv + b_ref[...], 0.0)
    out_ref[...] = lax.dot(h * dinv, w_ref[...],
                           preferred_element_type=jnp.float32)


def _tc_d_body(p_ref, g_ref, dinv_ref, b_ref, out_ref):
    agg = p_ref[0, :, 0:1] + p_ref[1, :, 0:1] + g_ref[:, 0:1]
    out_ref[...] = agg * dinv_ref[...] + b_ref[...]


def _row_block(d):
    return pl.BlockSpec((_B, d), lambda i: (i, 0))


def _part_block(d):
    return pl.BlockSpec((NC, _B, d), lambda i: (0, i, 0))


def _const_block(r, c):
    return pl.BlockSpec((r, c), lambda i: (0, 0))


_GRID = (N // _B,)

_tc_a = pl.pallas_call(
    _tc_a_body,
    grid=_GRID,
    in_specs=[_part_block(8), _row_block(128), _const_block(128, 64)],
    out_specs=[_row_block(64), _row_block(1)],
    out_shape=[jax.ShapeDtypeStruct((N, 64), jnp.float32),
               jax.ShapeDtypeStruct((N, 1), jnp.float32)],
)


def _tc_mid(dout):
    return pl.pallas_call(
        _tc_mid_body,
        grid=_GRID,
        in_specs=[_part_block(64), _row_block(64), _row_block(1),
                  _const_block(1, 64), _const_block(64, dout)],
        out_specs=_row_block(dout),
        out_shape=jax.ShapeDtypeStruct((N, dout), jnp.float32),
    )


_tc_b = _tc_mid(64)
_tc_c = _tc_mid(8)

_tc_d = pl.pallas_call(
    _tc_d_body,
    grid=_GRID,
    in_specs=[_part_block(8), _row_block(8), _row_block(1),
              _const_block(1, 1)],
    out_specs=_row_block(1),
    out_shape=jax.ShapeDtypeStruct((N, 1), jnp.float32),
)


def kernel(x, edge_index, W1, b1, W2, b2, W3, b3):
    ei = edge_index.astype(jnp.int32)
    zeros64 = jnp.zeros((NPAD, 64), jnp.float32)
    zeros8 = jnp.zeros((NPAD, 8), jnp.float32)
    ones8 = jnp.ones((K, 8), jnp.float32)

    pdeg = _deg8(ones8, ei, zeros8)
    g1, dinv = _tc_a(pdeg, x, W1)
    p1 = _agg64(g1, ei, zeros64)
    g2 = _tc_b(p1, g1, dinv, b1.reshape(1, 64), W2)
    p2 = _agg64(g2, ei, zeros64)
    W3_8 = jnp.tile(W3, (1, 8))
    g3 = _tc_c(p2, g2, dinv, b2.reshape(1, 64), W3_8)
    p3 = _agg8(g3, ei, zeros8)
    return _tc_d(p3, g3, dinv, b3.reshape(1, 1))


# speedup vs baseline: 1.3022x; 1.0220x over previous
import functools

import jax
import jax.numpy as jnp
from jax import lax
from jax.experimental import pallas as pl
from jax.experimental.pallas import tpu as pltpu
from jax.experimental.pallas import tpu_sc as plsc

N = 10000
E = 320000
NC = 2
NS = 16
NW = NC * NS
EPW = E // NW
K = 200
NCHUNK = EPW // K
NBUF = 5
NGROUP = NCHUNK // NBUF
NPAD = 10240
RPT = NPAD // NS

_MESH = plsc.VectorSubcoreMesh(core_axis_name="c", subcore_axis_name="s",
                               num_cores=NC, num_subcores=NS)


def _make_edge_agg(D, with_gather):
    def body(g_hbm, ei_hbm, zeros_hbm, out_hbm,
             src_slab, dst_slab, rows, acc, gsems, ssems):
        cid = lax.axis_index("c")
        sid = lax.axis_index("s")
        w = cid * NS + sid
        if with_gather:
            pltpu.sync_copy(ei_hbm.at[0, pl.ds(w * EPW, EPW)], src_slab)
        else:
            pltpu.sync_copy(g_hbm, rows.at[0])
        pltpu.sync_copy(ei_hbm.at[1, pl.ds(w * EPW, EPW)], dst_slab)
        if with_gather:
            @pl.when(cid == 0)
            def _():
                @pl.when(sid < NS - 1)
                def _():
                    pltpu.sync_copy(g_hbm.at[pl.ds(sid * RPT, RPT)],
                                    acc.at[pl.ds(sid * RPT, RPT)])

                @pl.when(sid == NS - 1)
                def _():
                    pltpu.sync_copy(g_hbm.at[pl.ds((NS - 1) * RPT, N - (NS - 1) * RPT)],
                                    acc.at[pl.ds((NS - 1) * RPT, N - (NS - 1) * RPT)])
                    pltpu.sync_copy(zeros_hbm.at[pl.ds(N, NPAD - N)],
                                    acc.at[pl.ds(N, NPAD - N)])

            @pl.when(cid == 1)
            def _():
                pltpu.sync_copy(zeros_hbm.at[pl.ds(sid * RPT, RPT)],
                                acc.at[pl.ds(sid * RPT, RPT)])
        else:
            pltpu.sync_copy(zeros_hbm.at[pl.ds(sid * RPT, RPT)],
                            acc.at[pl.ds(sid * RPT, RPT)])
        plsc.subcore_barrier()

        if with_gather:
            for b in range(NBUF):
                pltpu.async_copy(g_hbm.at[src_slab.at[pl.ds(b * K, K)]],
                                 rows.at[b], gsems[b])

            @pl.loop(0, NGROUP)
            def _(g):
                j0 = g * NBUF
                scats = []
                for b in range(NBUF):
                    pltpu.make_async_copy(
                        g_hbm.at[src_slab.at[pl.ds((j0 + b) * K, K)]],
                        rows.at[b], gsems[b]).wait()
                    scats.append(pltpu.async_copy(
                        rows.at[b], acc.at[dst_slab.at[pl.ds((j0 + b) * K, K)]],
                        ssems[b], add=True))
                for b in range(NBUF):
                    scats[b].wait()

                    @pl.when(j0 + NBUF + b < NCHUNK)
                    def _():
                        pltpu.async_copy(
                            g_hbm.at[src_slab.at[pl.ds((j0 + NBUF + b) * K, K)]],
                            rows.at[b], gsems[b])
        else:
            @pl.loop(0, NGROUP)
            def _(g):
                j0 = g * NBUF
                scats = [pltpu.async_copy(
                             rows.at[0],
                             acc.at[dst_slab.at[pl.ds((j0 + b) * K, K)]],
                             ssems[b], add=True)
                         for b in range(NBUF)]
                for s in scats:
                    s.wait()

        plsc.subcore_barrier()
        pltpu.sync_copy(acc.at[pl.ds(sid * RPT, RPT)],
                        out_hbm.at[cid, pl.ds(sid * RPT, RPT)])

    return pl.kernel(
        body,
        out_type=jax.ShapeDtypeStruct((NC, NPAD, D), jnp.float32),
        mesh=_MESH,
        scratch_types=[
            pltpu.VMEM((EPW,), jnp.int32),
            pltpu.VMEM((EPW,), jnp.int32),
            pltpu.VMEM((NBUF, K, D), jnp.float32),
            pltpu.VMEM_SHARED((NPAD, D), jnp.float32),
            [pltpu.SemaphoreType.DMA] * NBUF,
            [pltpu.SemaphoreType.DMA] * NBUF,
        ],
        compiler_params=pltpu.CompilerParams(use_tc_tiling_on_sc=False),
    )


_agg64 = _make_edge_agg(64, True)
_agg8 = _make_edge_agg(8, True)
_deg8 = _make_edge_agg(8, False)

_B = 5000


def _tc_a_body(pdeg_ref, x_ref, w1_ref, g1_ref, dinv_ref):
    deg = pdeg_ref[0, :, 0:1] + pdeg_ref[1, :, 0:1] + 1.0
    dinv = lax.rsqrt(deg)
    dinv_ref[...] = dinv
    g1_ref[...] = lax.dot(x_ref[...] * dinv, w1_ref[...],
                          preferred_element_type=jnp.float32)


def _tc_mid_body(p_ref, dinv_ref, b_ref, w_ref, out_ref):
    dinv = dinv_ref[...]
    agg = p_ref[0] + p_ref[1]
    h = jnp.maximum(agg * dinv + b_ref[...], 0.0)
    out_ref[...] = lax.dot(h * dinv, w_ref[...],
                           preferred_element_type=jnp.float32)


def _tc_d_body(p_ref, dinv_ref, b_ref, out_ref):
    agg = p_ref[0, :, 0:1] + p_ref[1, :, 0:1]
    out_ref[...] = agg * dinv_ref[...] + b_ref[...]


def _row_block(d):
    return pl.BlockSpec((_B, d), lambda i: (i, 0))


def _part_block(d):
    return pl.BlockSpec((NC, _B, d), lambda i: (0, i, 0))


def _const_block(r, c):
    return pl.BlockSpec((r, c), lambda i: (0, 0))


_GRID = (N // _B,)

_tc_a = pl.pallas_call(
    _tc_a_body,
    grid=_GRID,
    in_specs=[_part_block(8), _row_block(128), _const_block(128, 64)],
    out_specs=[_row_block(64), _row_block(1)],
    out_shape=[jax.ShapeDtypeStruct((N, 64), jnp.float32),
               jax.ShapeDtypeStruct((N, 1), jnp.float32)],
)


def _tc_mid(dout):
    return pl.pallas_call(
        _tc_mid_body,
        grid=_GRID,
        in_specs=[_part_block(64), _row_block(1),
                  _const_block(1, 64), _const_block(64, dout)],
        out_specs=_row_block(dout),
        out_shape=jax.ShapeDtypeStruct((N, dout), jnp.float32),
    )


_tc_b = _tc_mid(64)
_tc_c = _tc_mid(8)

_tc_d = pl.pallas_call(
    _tc_d_body,
    grid=_GRID,
    in_specs=[_part_block(8), _row_block(1),
              _const_block(1, 1)],
    out_specs=_row_block(1),
    out_shape=jax.ShapeDtypeStruct((N, 1), jnp.float32),
)


def kernel(x, edge_index, W1, b1, W2, b2, W3, b3):
    ei = edge_index.astype(jnp.int32)
    zeros64 = jnp.zeros((NPAD, 64), jnp.float32)
    zeros8 = jnp.zeros((NPAD, 8), jnp.float32)
    ones8 = jnp.ones((K, 8), jnp.float32)

    pdeg = _deg8(ones8, ei, zeros8)
    g1, dinv = _tc_a(pdeg, x, W1)
    p1 = _agg64(g1, ei, zeros64)
    g2 = _tc_b(p1, dinv, b1.reshape(1, 64), W2)
    p2 = _agg64(g2, ei, zeros64)
    W3_8 = jnp.tile(W3, (1, 8))
    g3 = _tc_c(p2, dinv, b2.reshape(1, 64), W3_8)
    p3 = _agg8(g3, ei, zeros8)
    return _tc_d(p3, dinv, b3.reshape(1, 1))


# speedup vs baseline: 1.4913x; 1.1453x over previous
import functools

import jax
import jax.numpy as jnp
from jax import lax
from jax.experimental import pallas as pl
from jax.experimental.pallas import tpu as pltpu
from jax.experimental.pallas import tpu_sc as plsc

N = 10000
E = 320000
NC = 2
NS = 16
NW = NC * NS
EPW = E // NW
K = 200
NCHUNK = EPW // K
NBUF = 5
NGROUP = NCHUNK // NBUF
NPAD = 10240
RPT = NPAD // NS

_MESH = plsc.VectorSubcoreMesh(core_axis_name="c", subcore_axis_name="s",
                               num_cores=NC, num_subcores=NS)


def _make_edge_agg(D, with_gather):
    def body(g_hbm, ei_hbm, zeros_hbm, out_hbm,
             src_slab, dst_slab, rows, acc, gsems, ssems):
        cid = lax.axis_index("c")
        sid = lax.axis_index("s")
        w = cid * NS + sid
        if with_gather:
            pltpu.sync_copy(ei_hbm.at[0, pl.ds(w * EPW, EPW)], src_slab)
        else:
            pltpu.sync_copy(g_hbm, rows.at[0])
        pltpu.sync_copy(ei_hbm.at[1, pl.ds(w * EPW, EPW)], dst_slab)
        if with_gather:
            @pl.when(cid == 0)
            def _():
                @pl.when(sid < NS - 1)
                def _():
                    pltpu.sync_copy(g_hbm.at[pl.ds(sid * RPT, RPT)],
                                    acc.at[pl.ds(sid * RPT, RPT)])

                @pl.when(sid == NS - 1)
                def _():
                    pltpu.sync_copy(g_hbm.at[pl.ds((NS - 1) * RPT, N - (NS - 1) * RPT)],
                                    acc.at[pl.ds((NS - 1) * RPT, N - (NS - 1) * RPT)])
                    pltpu.sync_copy(zeros_hbm.at[pl.ds(N, NPAD - N)],
                                    acc.at[pl.ds(N, NPAD - N)])

            @pl.when(cid == 1)
            def _():
                pltpu.sync_copy(zeros_hbm.at[pl.ds(sid * RPT, RPT)],
                                acc.at[pl.ds(sid * RPT, RPT)])
        else:
            pltpu.sync_copy(zeros_hbm.at[pl.ds(sid * RPT, RPT)],
                            acc.at[pl.ds(sid * RPT, RPT)])
        plsc.subcore_barrier()

        if with_gather:
            for b in range(NBUF):
                pltpu.async_copy(g_hbm.at[src_slab.at[pl.ds(b * K, K)]],
                                 rows.at[b], gsems[b])

            @pl.loop(0, NGROUP)
            def _(g):
                j0 = g * NBUF
                scats = []
                for b in range(NBUF):
                    pltpu.make_async_copy(
                        g_hbm.at[src_slab.at[pl.ds((j0 + b) * K, K)]],
                        rows.at[b], gsems[b]).wait()
                    scats.append(pltpu.async_copy(
                        rows.at[b], acc.at[dst_slab.at[pl.ds((j0 + b) * K, K)]],
                        ssems[b], add=True))
                for b in range(NBUF):
                    scats[b].wait()

                    @pl.when(j0 + NBUF + b < NCHUNK)
                    def _():
                        pltpu.async_copy(
                            g_hbm.at[src_slab.at[pl.ds((j0 + NBUF + b) * K, K)]],
                            rows.at[b], gsems[b])
        else:
            @pl.loop(0, NGROUP)
            def _(g):
                j0 = g * NBUF
                scats = [pltpu.async_copy(
                             rows.at[0],
                             acc.at[dst_slab.at[pl.ds((j0 + b) * K, K)]],
                             ssems[b], add=True)
                         for b in range(NBUF)]
                for s in scats:
                    s.wait()

        plsc.subcore_barrier()
        pltpu.sync_copy(acc.at[pl.ds(sid * RPT, RPT)],
                        out_hbm.at[pl.ds(sid * RPT, RPT), pl.ds(cid * 64, D)])

    return pl.kernel(
        body,
        out_type=jax.ShapeDtypeStruct((NPAD, 128), jnp.float32),
        mesh=_MESH,
        scratch_types=[
            pltpu.VMEM((EPW,), jnp.int32),
            pltpu.VMEM((EPW,), jnp.int32),
            pltpu.VMEM((NBUF, K, D), jnp.float32),
            pltpu.VMEM_SHARED((NPAD, D), jnp.float32),
            [pltpu.SemaphoreType.DMA] * NBUF,
            [pltpu.SemaphoreType.DMA] * NBUF,
        ],
        compiler_params=pltpu.CompilerParams(use_tc_tiling_on_sc=False),
    )


_agg64 = _make_edge_agg(64, True)
_agg8 = _make_edge_agg(8, True)
_deg8 = _make_edge_agg(8, False)

_B = 5000


def _tc_a_body(pdeg_ref, x_ref, w1_ref, g1_ref, dinv_ref):
    deg = pdeg_ref[:, 0:1] + pdeg_ref[:, 64:65] + 1.0
    dinv = lax.rsqrt(deg)
    dinv_ref[...] = dinv
    g1_ref[...] = lax.dot(x_ref[...] * dinv, w1_ref[...],
                          preferred_element_type=jnp.float32)


def _tc_mid_body(p_ref, dinv_ref, b_ref, w_ref, out_ref):
    dinv = dinv_ref[...]
    agg = p_ref[:, 0:64] + p_ref[:, 64:128]
    h = jnp.maximum(agg * dinv + b_ref[...], 0.0)
    out_ref[...] = lax.dot(h * dinv, w_ref[...],
                           preferred_element_type=jnp.float32)


def _tc_d_body(p_ref, dinv_ref, b_ref, out_ref):
    agg = p_ref[:, 0:1] + p_ref[:, 64:65]
    out_ref[...] = agg * dinv_ref[...] + b_ref[...]


def _row_block(d):
    return pl.BlockSpec((_B, d), lambda i: (i, 0))


def _part_block(d):
    return pl.BlockSpec((NC, _B, d), lambda i: (0, i, 0))


def _const_block(r, c):
    return pl.BlockSpec((r, c), lambda i: (0, 0))


_GRID = (N // _B,)

_tc_a = pl.pallas_call(
    _tc_a_body,
    grid=_GRID,
    in_specs=[_row_block(128), _row_block(128), _const_block(128, 64)],
    out_specs=[_row_block(64), _row_block(1)],
    out_shape=[jax.ShapeDtypeStruct((N, 64), jnp.float32),
               jax.ShapeDtypeStruct((N, 1), jnp.float32)],
)


def _tc_mid(dout):
    return pl.pallas_call(
        _tc_mid_body,
        grid=_GRID,
        in_specs=[_row_block(128), _row_block(1),
                  _const_block(1, 64), _const_block(64, dout)],
        out_specs=_row_block(dout),
        out_shape=jax.ShapeDtypeStruct((N, dout), jnp.float32),
    )


_tc_b = _tc_mid(64)
_tc_c = _tc_mid(8)

_tc_d = pl.pallas_call(
    _tc_d_body,
    grid=_GRID,
    in_specs=[_row_block(128), _row_block(1),
              _const_block(1, 1)],
    out_specs=_row_block(1),
    out_shape=jax.ShapeDtypeStruct((N, 1), jnp.float32),
)


def kernel(x, edge_index, W1, b1, W2, b2, W3, b3):
    ei = edge_index.astype(jnp.int32)
    zeros64 = jnp.zeros((NPAD, 64), jnp.float32)
    zeros8 = jnp.zeros((NPAD, 8), jnp.float32)
    ones8 = jnp.ones((K, 8), jnp.float32)

    pdeg = _deg8(ones8, ei, zeros8)
    g1, dinv = _tc_a(pdeg, x, W1)
    p1 = _agg64(g1, ei, zeros64)
    g2 = _tc_b(p1, dinv, b1.reshape(1, 64), W2)
    p2 = _agg64(g2, ei, zeros64)
    W3_8 = jnp.tile(W3, (1, 8))
    g3 = _tc_c(p2, dinv, b2.reshape(1, 64), W3_8)
    p3 = _agg8(g3, ei, zeros8)
    return _tc_d(p3, dinv, b3.reshape(1, 1))
